# baseline (speedup 1.0000x reference)
import functools

import jax
import jax.numpy as jnp
from jax import lax
from jax.experimental import pallas as pl
from jax.experimental.pallas import tpu as pltpu
from jax.experimental.pallas import tpu_sc as plsc


@functools.lru_cache(maxsize=None)
def _make_gather(V, D, S0, S1, C, NBUF):
    B = S0 * S1
    info = plsc.get_sparse_core_info()
    nc, ns = info.num_cores, info.num_subcores
    nw = nc * ns
    assert B % nw == 0
    b_per_w = B // nw
    assert b_per_w % (C * NBUF) == 0
    assert C == S1
    n_chunks = b_per_w // C
    n_outer = n_chunks // NBUF

    mesh = plsc.VectorSubcoreMesh(core_axis_name="c", subcore_axis_name="s")

    @functools.partial(
        pl.kernel,
        out_type=jax.ShapeDtypeStruct((S0, S1, D), jnp.float32),
        mesh=mesh,
        scratch_types=[
            pltpu.VMEM((b_per_w,), jnp.int32),
            pltpu.VMEM((NBUF, C, D), jnp.float32),
            pltpu.SemaphoreType.DMA((NBUF,)),
            pltpu.SemaphoreType.DMA((NBUF,)),
        ],
        compiler_params=pltpu.CompilerParams(use_tc_tiling_on_sc=False),
    )
    def k(table, idx_hbm, out_hbm, idx_v, rows, gsem, wsem):
        wid = lax.axis_index("s") * nc + lax.axis_index("c")
        base = wid * b_per_w
        s0_base = wid * (b_per_w // S1)
        pltpu.sync_copy(idx_hbm.at[pl.ds(base, b_per_w)], idx_v)

        def gather_start(j, b):
            pltpu.async_copy(
                table.at[idx_v.at[pl.ds(j * C, C)]], rows.at[b], gsem.at[b]
            )

        def gather_wait(b):
            pltpu.make_async_copy(
                table.at[idx_v.at[pl.ds(0, C)]], rows.at[b], gsem.at[b]
            ).wait()

        def write_start(j, b):
            pltpu.async_copy(rows.at[b], out_hbm.at[s0_base + j], wsem.at[b])

        def write_wait(b):
            pltpu.make_async_copy(
                rows.at[b], out_hbm.at[s0_base], wsem.at[b]
            ).wait()

        for b in range(NBUF):
            gather_start(b, b)

        def body(i, carry):
            for b in range(NBUF):
                j = i * NBUF + b
                gather_wait(b)
                write_start(j, b)
                write_wait(b)
                gather_start(j + NBUF, b)
            return carry

        lax.fori_loop(0, n_outer - 1, body, 0)

        for b in range(NBUF):
            j = (n_outer - 1) * NBUF + b
            gather_wait(b)
            write_start(j, b)
            write_wait(b)

    return k


def kernel(x, lut_weight):
    S0, S1 = x.shape
    V, D = lut_weight.shape
    B = S0 * S1
    idx = x.reshape(B).astype(jnp.int32)
    return _make_gather(V, D, S0, S1, S1, 8)(lut_weight, idx)


# speedup vs baseline: 1.3310x; 1.3310x over previous
import functools

import jax
import jax.numpy as jnp
from jax import lax
from jax.experimental import pallas as pl
from jax.experimental.pallas import tpu as pltpu
from jax.experimental.pallas import tpu_sc as plsc


@functools.lru_cache(maxsize=None)
def _make_gather(V, D, S0, S1, C, NBUF):
    B = S0 * S1
    info = plsc.get_sparse_core_info()
    nc, ns = info.num_cores, info.num_subcores
    nw = nc * ns
    assert B % nw == 0
    b_per_w = B // nw
    assert b_per_w % (C * NBUF) == 0
    assert C == S1
    n_chunks = b_per_w // C
    n_outer = n_chunks // NBUF

    mesh = plsc.VectorSubcoreMesh(core_axis_name="c", subcore_axis_name="s")

    @functools.partial(
        pl.kernel,
        out_type=jax.ShapeDtypeStruct((B, 2 * D), jnp.float32),
        mesh=mesh,
        scratch_types=[
            pltpu.VMEM((b_per_w,), jnp.int32),
            pltpu.VMEM((NBUF, C, D), jnp.float32),
            pltpu.SemaphoreType.DMA((NBUF,)),
            pltpu.SemaphoreType.DMA((NBUF,)),
        ],
        compiler_params=pltpu.CompilerParams(use_tc_tiling_on_sc=False),
    )
    def k(table, idx_hbm, out_hbm, idx_v, rows, gsem, wsem):
        wid = lax.axis_index("s") * nc + lax.axis_index("c")
        base = wid * b_per_w
        pltpu.sync_copy(idx_hbm.at[pl.ds(base, b_per_w)], idx_v)

        def gather_start(j, b):
            pltpu.async_copy(
                table.at[idx_v.at[pl.ds(j * C, C)]], rows.at[b], gsem.at[b]
            )

        def gather_wait(b):
            pltpu.make_async_copy(
                table.at[idx_v.at[pl.ds(0, C)]], rows.at[b], gsem.at[b]
            ).wait()

        def write_start(j, b):
            pltpu.async_copy(
                rows.at[b],
                out_hbm.at[pl.ds(base + j * C, C), pl.ds(0, D)],
                wsem.at[b],
            )

        def write_wait(b):
            pltpu.make_async_copy(
                rows.at[b], out_hbm.at[pl.ds(base, C), pl.ds(0, D)], wsem.at[b]
            ).wait()

        for b in range(NBUF):
            gather_start(b, b)

        def body(i, carry):
            for b in range(NBUF):
                j = i * NBUF + b
                gather_wait(b)
                write_start(j, b)
                write_wait(b)
                gather_start(j + NBUF, b)
            return carry

        lax.fori_loop(0, n_outer - 1, body, 0)

        for b in range(NBUF):
            j = (n_outer - 1) * NBUF + b
            gather_wait(b)
            write_start(j, b)
            write_wait(b)

    return k


def kernel(x, lut_weight):
    S0, S1 = x.shape
    V, D = lut_weight.shape
    B = S0 * S1
    idx = x.reshape(B).astype(jnp.int32)
    out = _make_gather(V, D, S0, S1, S1, 8)(lut_weight, idx)
    return out[:, :D].reshape(S0, S1, D)
